# baseline (speedup 1.0000x reference)
import functools

import jax
import jax.numpy as jnp
from jax import lax
from jax.experimental import pallas as pl
from jax.experimental.pallas import tpu as pltpu
from jax.experimental.pallas import tpu_sc as plsc

VOCAB = 1000000
DIM = 64
BATCH = 16384

_NUM_WORKERS = 32
_RPW = BATCH // _NUM_WORKERS
_L = 16
_G = 4
_NG = _RPW // _G


@functools.partial(
    pl.kernel,
    mesh=plsc.VectorSubcoreMesh(core_axis_name="c", subcore_axis_name="s"),
    out_type=jax.ShapeDtypeStruct((DIM, BATCH), jnp.float32),
    scratch_types=[
        pltpu.VMEM((_RPW + 16,), jnp.int32),
        pltpu.VMEM((DIM, 3 * _G * 128), jnp.float32),
        pltpu.VMEM((DIM, 128), jnp.float32),
        pltpu.SemaphoreType.DMA,
    ],
    compiler_params=pltpu.CompilerParams(needs_layout_passes=False),
)
def _gather_kernel(idx_hbm, table_hbm, out_hbm, idx_v, ring_v, cols_v, sem):
    wid = lax.axis_index("s") * 2 + lax.axis_index("c")
    base = wid * _RPW
    pltpu.sync_copy(idx_hbm.at[pl.ds(base, _RPW)], idx_v.at[pl.ds(0, _RPW)])

    lane = lax.iota(jnp.int32, _L)

    def fire(g, par):
        vec = idx_v[pl.ds(g * _G, _L)]
        panel = lax.shift_right_logical(vec, 7)
        for r in range(_G):
            c0 = pl.multiple_of(panel[r] * 128, 128)
            pltpu.make_async_copy(
                table_hbm.at[:, pl.ds(c0, 128)],
                ring_v.at[:, pl.ds((par * _G + r) * 128, 128)],
                sem,
            ).start()

    def drain(par):
        pltpu.make_async_copy(
            table_hbm.at[:, pl.ds(0, _G * 128)],
            ring_v.at[:, pl.ds(par * _G * 128, _G * 128)],
            sem,
        ).wait()

    def extract(g, par):
        vec = idx_v[pl.ds(g * _G, _L)]
        col = lax.bitwise_and(vec, 127)
        for r in range(_G):
            vcol = jnp.full((_L,), (par * _G + r) * 128, jnp.int32) + col[r]
            jcol = jnp.full((_L,), 0, jnp.int32) + ((g * _G + r) & 127)
            for k in range(DIM // _L):
                d = lane + k * _L
                v = plsc.load_gather(ring_v, [d, vcol])
                plsc.store_scatter(cols_v, [d, jcol], v)
        @pl.when((g & 31) == 31)
        def _():
            j0 = pl.multiple_of(g * _G - 124, 128)
            pltpu.sync_copy(cols_v, out_hbm.at[:, pl.ds(base + j0, 128)])

    fire(0, 0)
    fire(1, 1)

    def step(si, _):
        for sub in range(3):
            g = si * 3 + sub
            fire(g + 2, (sub + 2) % 3)
            drain(sub)
            extract(g, sub)
        return 0

    lax.fori_loop(0, (_NG - 2) // 3, step, 0)
    drain(0)
    extract(_NG - 2, 0)
    drain(1)
    extract(_NG - 1, 1)


def kernel(indices, table):
    out_t = _gather_kernel(indices, table.T)
    return out_t.T


# speedup vs baseline: 1.0110x; 1.0110x over previous
import functools

import jax
import jax.numpy as jnp
from jax import lax
from jax.experimental import pallas as pl
from jax.experimental.pallas import tpu as pltpu
from jax.experimental.pallas import tpu_sc as plsc

VOCAB = 1000000
DIM = 64
BATCH = 16384

_NUM_WORKERS = 32
_RPW = BATCH // _NUM_WORKERS
_L = 16
_G = 4
_NG = _RPW // _G


@functools.partial(
    pl.kernel,
    mesh=plsc.VectorSubcoreMesh(core_axis_name="c", subcore_axis_name="s"),
    out_type=jax.ShapeDtypeStruct((DIM, BATCH), jnp.float32),
    scratch_types=[
        pltpu.VMEM((_RPW + 16,), jnp.int32),
        pltpu.VMEM((DIM, 3 * _G * 128), jnp.float32),
        pltpu.VMEM((DIM, 128), jnp.float32),
        pltpu.SemaphoreType.DMA,
    ],
    compiler_params=pltpu.CompilerParams(needs_layout_passes=False),
)
def _gather_kernel(idx_hbm, table_hbm, out_hbm, idx_v, ring_v, cols_v, sem):
    wid = lax.axis_index("s") * 2 + lax.axis_index("c")
    base = wid * _RPW
    pltpu.sync_copy(idx_hbm.at[pl.ds(base, _RPW)], idx_v.at[pl.ds(0, _RPW)])

    lane = lax.iota(jnp.int32, _L)

    def fire(g, par):
        vec = idx_v[pl.ds(g * _G, _L)]
        panel = lax.shift_right_logical(vec, 7)
        for r in range(_G):
            c0 = pl.multiple_of(panel[r] * 128, 128)
            pltpu.make_async_copy(
                table_hbm.at[:, pl.ds(c0, 128)],
                ring_v.at[:, pl.ds((par * _G + r) * 128, 128)],
                sem,
            ).start()

    def drain(par):
        pltpu.make_async_copy(
            table_hbm.at[:, pl.ds(0, _G * 128)],
            ring_v.at[:, pl.ds(par * _G * 128, _G * 128)],
            sem,
        ).wait()

    def extract(g, par):
        vec = idx_v[pl.ds(g * _G, _L)]
        col = lax.bitwise_and(vec, 127)
        for r in range(_G):
            vcol = jnp.full((_L,), (par * _G + r) * 128, jnp.int32) + col[r]
            jcol = jnp.full((_L,), 0, jnp.int32) + ((g * _G + r) & 127)
            for k in range(DIM // _L):
                d = lane + k * _L
                v = plsc.load_gather(ring_v, [d, vcol])
                plsc.store_scatter(cols_v, [d, jcol], v)
        @pl.when((g & 31) == 31)
        def _():
            j0 = pl.multiple_of(g * _G - 124, 128)
            pltpu.sync_copy(cols_v, out_hbm.at[:, pl.ds(base + j0, 128)])

    fire(0, 0)
    fire(1, 1)

    def step(si, _):
        for sub in range(3):
            g = si * 3 + sub
            drain(sub)
            fire(g + 2, (sub + 2) % 3)
            extract(g, sub)
        return 0

    lax.fori_loop(0, (_NG - 2) // 3, step, 0)
    drain(0)
    extract(_NG - 2, 0)
    drain(1)
    extract(_NG - 1, 1)


def kernel(indices, table):
    out_t = _gather_kernel(indices, table.T)
    return out_t.T


# speedup vs baseline: 1.0167x; 1.0056x over previous
import functools

import jax
import jax.numpy as jnp
from jax import lax
from jax.experimental import pallas as pl
from jax.experimental.pallas import tpu as pltpu
from jax.experimental.pallas import tpu_sc as plsc

VOCAB = 1000000
DIM = 64
BATCH = 16384

_NUM_WORKERS = 32
_RPW = BATCH // _NUM_WORKERS
_L = 16
_G = 4
_NG = _RPW // _G


@functools.partial(
    pl.kernel,
    mesh=plsc.VectorSubcoreMesh(core_axis_name="c", subcore_axis_name="s"),
    out_type=jax.ShapeDtypeStruct((DIM, BATCH), jnp.float32),
    scratch_types=[
        pltpu.VMEM((_RPW + 16,), jnp.int32),
        pltpu.VMEM((DIM, 3 * _G * 128), jnp.float32),
        pltpu.VMEM((DIM, 128), jnp.float32),
        pltpu.SemaphoreType.DMA,
    ],
    compiler_params=pltpu.CompilerParams(needs_layout_passes=False),
)
def _gather_kernel(idx_hbm, table_hbm, out_hbm, idx_v, ring_v, cols_v, sem):
    wid = lax.axis_index("s") * 2 + lax.axis_index("c")
    base = wid * _RPW
    pltpu.sync_copy(idx_hbm.at[pl.ds(base, _RPW)], idx_v.at[pl.ds(0, _RPW)])

    lane = lax.iota(jnp.int32, _L)

    def fire(g, par):
        vec = idx_v[pl.ds(g * _G, _L)]
        panel = lax.shift_right_logical(vec, 7)
        for r in range(_G):
            c0 = pl.multiple_of(panel[r] * 128, 128)
            for t in range(8):
                pltpu.make_async_copy(
                    table_hbm.at[pl.ds(t * 8, 8), pl.ds(c0, 128)],
                    ring_v.at[pl.ds(t * 8, 8),
                              pl.ds((par * _G + r) * 128, 128)],
                    sem,
                ).start()

    def drain(par):
        pltpu.make_async_copy(
            table_hbm.at[:, pl.ds(0, _G * 128)],
            ring_v.at[:, pl.ds(par * _G * 128, _G * 128)],
            sem,
        ).wait()

    def extract(g, par):
        vec = idx_v[pl.ds(g * _G, _L)]
        col = lax.bitwise_and(vec, 127)
        for r in range(_G):
            vcol = jnp.full((_L,), (par * _G + r) * 128, jnp.int32) + col[r]
            jcol = jnp.full((_L,), 0, jnp.int32) + ((g * _G + r) & 127)
            for k in range(DIM // _L):
                d = lane + k * _L
                v = plsc.load_gather(ring_v, [d, vcol])
                plsc.store_scatter(cols_v, [d, jcol], v)
        @pl.when((g & 31) == 31)
        def _():
            j0 = pl.multiple_of(g * _G - 124, 128)
            pltpu.sync_copy(cols_v, out_hbm.at[:, pl.ds(base + j0, 128)])

    fire(0, 0)
    fire(1, 1)

    def step(si, _):
        for sub in range(3):
            g = si * 3 + sub
            drain(sub)
            fire(g + 2, (sub + 2) % 3)
            extract(g, sub)
        return 0

    lax.fori_loop(0, (_NG - 2) // 3, step, 0)
    drain(0)
    extract(_NG - 2, 0)
    drain(1)
    extract(_NG - 1, 1)


def kernel(indices, table):
    out_t = _gather_kernel(indices, table.T)
    return out_t.T
